# Initial kernel scaffold; baseline (speedup 1.0000x reference)
import jax
import jax.numpy as jnp
from jax.experimental import pallas as pl


def kernel(x, Wg, bg, W1, b1, W2, b2):
    raise NotImplementedError("write your pallas kernel here")


# speedup vs baseline: 3.0669x; 3.0669x over previous
import jax
import jax.numpy as jnp
from jax.experimental import pallas as pl
from jax.experimental.pallas import tpu as pltpu

D = 768
H = 3072
E = 8
T = 2048
HB = 2
HBS = H // HB
CHUNK = 512
NCH = T // CHUNK


def _moe_kernel(x_ref, wg_ref, bg_ref, w1_ref, b1_ref, w2_ref, b2_ref,
                out_ref, wmat_ref):
    e = pl.program_id(0)
    j = pl.program_id(1)
    lane = jax.lax.broadcasted_iota(jnp.int32, (1, E), 1)

    @pl.when(jnp.logical_and(e == 0, j == 0))
    def _gating():
        logits = jnp.dot(x_ref[...].astype(jnp.bfloat16),
                         wg_ref[...].astype(jnp.bfloat16),
                         preferred_element_type=jnp.float32) + bg_ref[...]
        lane2 = jax.lax.broadcasted_iota(jnp.int32, logits.shape, 1)
        v1 = jnp.max(logits, axis=-1, keepdims=True)
        is1 = logits == v1
        i1 = jnp.min(jnp.where(is1, lane2, E), axis=-1, keepdims=True)
        oh1 = lane2 == i1
        masked = jnp.where(oh1, -jnp.inf, logits)
        v2 = jnp.max(masked, axis=-1, keepdims=True)
        is2 = masked == v2
        i2 = jnp.min(jnp.where(is2, lane2, E), axis=-1, keepdims=True)
        oh2 = lane2 == i2
        p1 = jax.nn.sigmoid(v1 - v2)
        p2 = jax.nn.sigmoid(v2 - v1)
        wmat_ref[...] = jnp.where(oh1, p1, 0.0) + jnp.where(oh2, p2, 0.0)

    w1 = w1_ref[0].astype(jnp.bfloat16)
    w2 = w2_ref[0].astype(jnp.bfloat16)
    b1 = b1_ref[0]
    bmul = (j == 0).astype(jnp.float32)
    first = jnp.logical_and(e == 0, j == 0)

    for c in range(NCH):
        rows = pl.ds(c * CHUNK, CHUNK)
        xb = x_ref[rows, :].astype(jnp.bfloat16)
        h = jnp.dot(xb, w1, preferred_element_type=jnp.float32) + b1
        h = 0.5 * h * (1.0 + jax.lax.erf(h * 0.7071067811865476))
        y = jnp.dot(h.astype(jnp.bfloat16), w2,
                    preferred_element_type=jnp.float32)
        wcol = jnp.sum(jnp.where(lane == e, wmat_ref[rows, :], 0.0),
                       axis=-1, keepdims=True)
        contrib = (y + bmul * b2_ref[0]) * wcol

        @pl.when(first)
        def _init():
            out_ref[rows, :] = contrib

        @pl.when(jnp.logical_not(first))
        def _acc():
            out_ref[rows, :] += contrib


def kernel(x, Wg, bg, W1, b1, W2, b2):
    bg2 = bg.reshape(1, E)
    b1r = b1.reshape(E, 1, H)
    b2r = b2.reshape(E, 1, D)
    return pl.pallas_call(
        _moe_kernel,
        grid=(E, HB),
        in_specs=[
            pl.BlockSpec((T, D), lambda e, j: (0, 0)),
            pl.BlockSpec((D, E), lambda e, j: (0, 0)),
            pl.BlockSpec((1, E), lambda e, j: (0, 0)),
            pl.BlockSpec((1, D, HBS), lambda e, j: (e, 0, j)),
            pl.BlockSpec((1, 1, HBS), lambda e, j: (e, 0, j)),
            pl.BlockSpec((1, HBS, D), lambda e, j: (e, j, 0)),
            pl.BlockSpec((1, 1, D), lambda e, j: (e, 0, 0)),
        ],
        out_specs=pl.BlockSpec((T, D), lambda e, j: (0, 0)),
        out_shape=jax.ShapeDtypeStruct((T, D), jnp.float32),
        scratch_shapes=[pltpu.VMEM((T, E), jnp.float32)],
    )(x, Wg, bg2, W1, b1r, W2, b2r)


# speedup vs baseline: 3.9047x; 1.2732x over previous
import functools

import jax
import jax.numpy as jnp
from jax import lax
from jax.experimental import pallas as pl
from jax.experimental.pallas import tpu as pltpu
from jax.experimental.pallas import tpu_sc as plsc

D = 768
H = 3072
E = 8
T = 2048
BT = 256
NTILES = 24
S_PAD = NTILES * BT
NC = 2
NS = 16
NW = NC * NS
TPW = T // NW
MLEN = 32


def _gate_kernel(x_ref, wg_ref, bg_ref,
                 pose_ref, poso_ref, w0_ref, w1_ref, meta_ref):
    logits = jnp.dot(x_ref[...].astype(jnp.bfloat16),
                     wg_ref[...].astype(jnp.bfloat16),
                     preferred_element_type=jnp.float32) + bg_ref[...]
    lane2 = lax.broadcasted_iota(jnp.int32, logits.shape, 1)
    v1 = jnp.max(logits, axis=-1, keepdims=True)
    i1 = jnp.min(jnp.where(logits == v1, lane2, E), axis=-1, keepdims=True)
    oh1 = lane2 == i1
    masked = jnp.where(oh1, -jnp.inf, logits)
    v2 = jnp.max(masked, axis=-1, keepdims=True)
    i2 = jnp.min(jnp.where(masked == v2, lane2, E), axis=-1, keepdims=True)
    oh2 = lane2 == i2
    w0_ref[...] = jax.nn.sigmoid(v1 - v2)
    w1_ref[...] = jax.nn.sigmoid(v2 - v1)

    both = (oh1 | oh2).astype(jnp.bfloat16)
    rr = lax.broadcasted_iota(jnp.int32, (T, T), 0)
    cc = lax.broadcasted_iota(jnp.int32, (T, T), 1)
    ltri = (cc < rr).astype(jnp.bfloat16)
    excl = jnp.dot(ltri, both, preferred_element_type=jnp.float32)
    counts = jnp.sum(both.astype(jnp.float32), axis=0, keepdims=True)
    pc = jnp.ceil(counts / BT) * BT
    lane8f = lax.broadcasted_iota(jnp.int32, (1, E), 1).astype(jnp.float32)
    cum_pc = jnp.zeros((1, E), jnp.float32)
    for e in range(E):
        ce = jnp.sum(jnp.where(lane8f <= e, pc, 0.0))
        cum_pc = jnp.where(lane8f == e, ce, cum_pc)
    goff = cum_pc - pc

    def pick(mat, idx):
        return jnp.sum(jnp.where(lane2 == idx, mat, 0.0),
                       axis=-1, keepdims=True)

    pose_ref[...] = (pick(goff + excl, i1)).astype(jnp.int32)
    poso_ref[...] = (pick(goff + excl, i2)).astype(jnp.int32)

    lane32 = lax.broadcasted_iota(jnp.int32, (1, MLEN), 1)
    lane8 = lax.broadcasted_iota(jnp.int32, (1, E), 1).astype(jnp.float32)
    acc = jnp.zeros((1, MLEN), jnp.int32)
    for e in range(E):
        cum_e = jnp.sum(jnp.where(lane8 == e, cum_pc, 0.0))
        acc = acc + (cum_e <= (lane32 * BT).astype(jnp.float32)).astype(jnp.int32)
    te = jnp.minimum(acc, E - 1)
    used = (jnp.sum(jnp.where(lane8 == E - 1, cum_pc, 0.0)) / BT).astype(jnp.int32)
    meta_ref[...] = jnp.where(lane32 < NTILES, te, used)


def _gate(x, Wg, bg):
    return pl.pallas_call(
        _gate_kernel,
        grid=(1,),
        in_specs=[
            pl.BlockSpec((T, D), lambda i: (0, 0)),
            pl.BlockSpec((D, E), lambda i: (0, 0)),
            pl.BlockSpec((1, E), lambda i: (0, 0)),
        ],
        out_specs=[
            pl.BlockSpec((T, 1), lambda i: (0, 0)),
            pl.BlockSpec((T, 1), lambda i: (0, 0)),
            pl.BlockSpec((T, 1), lambda i: (0, 0)),
            pl.BlockSpec((T, 1), lambda i: (0, 0)),
            pl.BlockSpec((1, MLEN), lambda i: (0, 0)),
        ],
        out_shape=[
            jax.ShapeDtypeStruct((T, 1), jnp.int32),
            jax.ShapeDtypeStruct((T, 1), jnp.int32),
            jax.ShapeDtypeStruct((T, 1), jnp.float32),
            jax.ShapeDtypeStruct((T, 1), jnp.float32),
            jax.ShapeDtypeStruct((1, MLEN), jnp.int32),
        ],
    )(x, Wg, bg.reshape(1, E))


def _sc_scatter_xs(x, pose3, poso3):
    mesh = plsc.VectorSubcoreMesh(core_axis_name="c", subcore_axis_name="s")

    @functools.partial(
        pl.kernel, mesh=mesh,
        out_type=jax.ShapeDtypeStruct((S_PAD, D), jnp.float32),
        scratch_types=[
            pltpu.VMEM((1, TPW), jnp.int32),
            pltpu.VMEM((1, TPW), jnp.int32),
            pltpu.VMEM((TPW, D), jnp.float32),
        ],
    )
    def k(x_hbm, pose_hbm, poso_hbm, xs_hbm, ie_v, io_v, rows_v):
        wid = lax.axis_index("s") * NC + lax.axis_index("c")
        pltpu.sync_copy(pose_hbm.at[wid], ie_v)
        pltpu.sync_copy(poso_hbm.at[wid], io_v)
        pltpu.sync_copy(x_hbm.at[pl.ds(wid * TPW, TPW)], rows_v)
        pltpu.sync_copy(rows_v, xs_hbm.at[ie_v.at[0]])
        pltpu.sync_copy(rows_v, xs_hbm.at[io_v.at[0]])

    return k(x, pose3, poso3)


def _ffn_kernel(meta_ref, xs_ref, w1_ref, b1_ref, w2_ref, b2_ref, ys_ref):
    t = pl.program_id(0)

    @pl.when(t < meta_ref[NTILES])
    def _():
        xb = xs_ref[...].astype(jnp.bfloat16)
        h = jnp.dot(xb, w1_ref[0].astype(jnp.bfloat16),
                    preferred_element_type=jnp.float32) + b1_ref[0]
        h = 0.5 * h * (1.0 + lax.erf(h * 0.7071067811865476))
        ys_ref[...] = jnp.dot(h.astype(jnp.bfloat16),
                              w2_ref[0].astype(jnp.bfloat16),
                              preferred_element_type=jnp.float32) + b2_ref[0]


def _ffn(meta, xs, W1, b1, W2, b2):
    grid_spec = pltpu.PrefetchScalarGridSpec(
        num_scalar_prefetch=1,
        grid=(NTILES,),
        in_specs=[
            pl.BlockSpec((BT, D), lambda t, m: (t, 0)),
            pl.BlockSpec((1, D, H), lambda t, m: (m[t], 0, 0)),
            pl.BlockSpec((1, 1, H), lambda t, m: (m[t], 0, 0)),
            pl.BlockSpec((1, H, D), lambda t, m: (m[t], 0, 0)),
            pl.BlockSpec((1, 1, D), lambda t, m: (m[t], 0, 0)),
        ],
        out_specs=pl.BlockSpec((BT, D), lambda t, m: (t, 0)),
    )
    return pl.pallas_call(
        _ffn_kernel,
        grid_spec=grid_spec,
        out_shape=jax.ShapeDtypeStruct((S_PAD, D), jnp.float32),
    )(meta, xs, W1, b1.reshape(E, 1, H), W2, b2.reshape(E, 1, D))


def _sc_gather_y(ys, pose3, poso3):
    mesh = plsc.VectorSubcoreMesh(core_axis_name="c", subcore_axis_name="s")
    oty = jax.ShapeDtypeStruct((T, D), jnp.float32)

    @functools.partial(
        pl.kernel, mesh=mesh,
        out_type=[oty, oty],
        scratch_types=[
            pltpu.VMEM((1, TPW), jnp.int32),
            pltpu.VMEM((1, TPW), jnp.int32),
            pltpu.VMEM((TPW, D), jnp.float32),
            pltpu.VMEM((TPW, D), jnp.float32),
        ],
    )
    def k(ys_hbm, pose_hbm, poso_hbm, y0_hbm, y1_hbm, ie_v, io_v, r0_v, r1_v):
        wid = lax.axis_index("s") * NC + lax.axis_index("c")
        pltpu.sync_copy(pose_hbm.at[wid], ie_v)
        pltpu.sync_copy(poso_hbm.at[wid], io_v)
        pltpu.sync_copy(ys_hbm.at[ie_v.at[0]], r0_v)
        pltpu.sync_copy(ys_hbm.at[io_v.at[0]], r1_v)
        pltpu.sync_copy(r0_v, y0_hbm.at[pl.ds(wid * TPW, TPW)])
        pltpu.sync_copy(r1_v, y1_hbm.at[pl.ds(wid * TPW, TPW)])

    return k(ys, pose3, poso3)


def _combine_kernel(y0_ref, y1_ref, w0_ref, w1_ref, out_ref):
    out_ref[...] = y0_ref[...] * w0_ref[...] + y1_ref[...] * w1_ref[...]


def _combine(y0, y1, w0, w1):
    return pl.pallas_call(
        _combine_kernel,
        grid=(4,),
        in_specs=[
            pl.BlockSpec((T // 4, D), lambda i: (i, 0)),
            pl.BlockSpec((T // 4, D), lambda i: (i, 0)),
            pl.BlockSpec((T // 4, 1), lambda i: (i, 0)),
            pl.BlockSpec((T // 4, 1), lambda i: (i, 0)),
        ],
        out_specs=pl.BlockSpec((T // 4, D), lambda i: (i, 0)),
        out_shape=jax.ShapeDtypeStruct((T, D), jnp.float32),
    )(y0, y1, w0, w1)


def kernel(x, Wg, bg, W1, b1, W2, b2):
    pose, poso, w0, w1, meta = _gate(x, Wg, bg)
    pose3 = pose.reshape(NW, 1, TPW)
    poso3 = poso.reshape(NW, 1, TPW)
    xs = _sc_scatter_xs(x, pose3, poso3)
    ys = _ffn(meta.reshape(MLEN), xs, W1, b1, W2, b2)
    y0, y1 = _sc_gather_y(ys, pose3, poso3)
    return _combine(y0, y1, w0, w1)


# speedup vs baseline: 4.0150x; 1.0283x over previous
import functools

import jax
import jax.numpy as jnp
from jax import lax
from jax.experimental import pallas as pl
from jax.experimental.pallas import tpu as pltpu
from jax.experimental.pallas import tpu_sc as plsc

D = 768
H = 3072
E = 8
T = 2048
BT = 256
NTILES = 24
S_PAD = NTILES * BT
NC = 2
NS = 16
NW = NC * NS
TPW = T // NW
MLEN = 32


def _gate_kernel(x_ref, wg_ref, bg_ref,
                 pose_ref, poso_ref, w0_ref, w1_ref, meta_ref):
    logits = jnp.dot(x_ref[...].astype(jnp.bfloat16),
                     wg_ref[...].astype(jnp.bfloat16),
                     preferred_element_type=jnp.float32) + bg_ref[...]
    lane2 = lax.broadcasted_iota(jnp.int32, logits.shape, 1)
    v1 = jnp.max(logits, axis=-1, keepdims=True)
    i1 = jnp.min(jnp.where(logits == v1, lane2, E), axis=-1, keepdims=True)
    oh1 = lane2 == i1
    masked = jnp.where(oh1, -jnp.inf, logits)
    v2 = jnp.max(masked, axis=-1, keepdims=True)
    i2 = jnp.min(jnp.where(masked == v2, lane2, E), axis=-1, keepdims=True)
    oh2 = lane2 == i2
    w0_ref[...] = jax.nn.sigmoid(v1 - v2)
    w1_ref[...] = jax.nn.sigmoid(v2 - v1)

    both = (oh1 | oh2).astype(jnp.bfloat16)
    rr = lax.broadcasted_iota(jnp.int32, (T, T), 0)
    cc = lax.broadcasted_iota(jnp.int32, (T, T), 1)
    ltri = (cc < rr).astype(jnp.bfloat16)
    excl = jnp.dot(ltri, both, preferred_element_type=jnp.float32)
    counts = jnp.sum(both.astype(jnp.float32), axis=0, keepdims=True)
    pc = jnp.ceil(counts / BT) * BT
    lane8f = lax.broadcasted_iota(jnp.int32, (1, E), 1).astype(jnp.float32)
    cum_pc = jnp.zeros((1, E), jnp.float32)
    for e in range(E):
        ce = jnp.sum(jnp.where(lane8f <= e, pc, 0.0))
        cum_pc = jnp.where(lane8f == e, ce, cum_pc)
    goff = cum_pc - pc

    def pick(mat, idx):
        return jnp.sum(jnp.where(lane2 == idx, mat, 0.0),
                       axis=-1, keepdims=True)

    pose_ref[...] = (pick(goff + excl, i1)).astype(jnp.int32)
    poso_ref[...] = (pick(goff + excl, i2)).astype(jnp.int32)

    lane32 = lax.broadcasted_iota(jnp.int32, (1, MLEN), 1)
    lane8 = lax.broadcasted_iota(jnp.int32, (1, E), 1).astype(jnp.float32)
    acc = jnp.zeros((1, MLEN), jnp.int32)
    for e in range(E):
        cum_e = jnp.sum(jnp.where(lane8 == e, cum_pc, 0.0))
        acc = acc + (cum_e <= (lane32 * BT).astype(jnp.float32)).astype(jnp.int32)
    te = jnp.minimum(acc, E - 1)
    used = (jnp.sum(jnp.where(lane8 == E - 1, cum_pc, 0.0)) / BT).astype(jnp.int32)
    meta_ref[...] = jnp.where(lane32 < NTILES, te, used)


def _gate(x, Wg, bg):
    return pl.pallas_call(
        _gate_kernel,
        grid=(1,),
        in_specs=[
            pl.BlockSpec((T, D), lambda i: (0, 0)),
            pl.BlockSpec((D, E), lambda i: (0, 0)),
            pl.BlockSpec((1, E), lambda i: (0, 0)),
        ],
        out_specs=[
            pl.BlockSpec((T, 1), lambda i: (0, 0)),
            pl.BlockSpec((T, 1), lambda i: (0, 0)),
            pl.BlockSpec((T, 1), lambda i: (0, 0)),
            pl.BlockSpec((T, 1), lambda i: (0, 0)),
            pl.BlockSpec((1, MLEN), lambda i: (0, 0)),
        ],
        out_shape=[
            jax.ShapeDtypeStruct((T, 1), jnp.int32),
            jax.ShapeDtypeStruct((T, 1), jnp.int32),
            jax.ShapeDtypeStruct((T, 1), jnp.float32),
            jax.ShapeDtypeStruct((T, 1), jnp.float32),
            jax.ShapeDtypeStruct((1, MLEN), jnp.int32),
        ],
    )(x, Wg, bg.reshape(1, E))


def _sc_scatter_xs(x, pose3, poso3):
    mesh = plsc.VectorSubcoreMesh(core_axis_name="c", subcore_axis_name="s")

    @functools.partial(
        pl.kernel, mesh=mesh,
        out_type=jax.ShapeDtypeStruct((S_PAD, D), jnp.float32),
        scratch_types=[
            pltpu.VMEM((1, TPW), jnp.int32),
            pltpu.VMEM((1, TPW), jnp.int32),
            pltpu.VMEM((TPW, D), jnp.float32),
            pltpu.SemaphoreType.DMA,
            pltpu.SemaphoreType.DMA,
            pltpu.SemaphoreType.DMA,
        ],
    )
    def k(x_hbm, pose_hbm, poso_hbm, xs_hbm, ie_v, io_v, rows_v, s0, s1, s2):
        wid = lax.axis_index("s") * NC + lax.axis_index("c")
        a0 = pltpu.async_copy(pose_hbm.at[wid], ie_v, s0)
        a1 = pltpu.async_copy(poso_hbm.at[wid], io_v, s1)
        a2 = pltpu.async_copy(x_hbm.at[pl.ds(wid * TPW, TPW)], rows_v, s2)
        a0.wait()
        a1.wait()
        a2.wait()
        b0 = pltpu.async_copy(rows_v, xs_hbm.at[ie_v.at[0]], s0)
        b1 = pltpu.async_copy(rows_v, xs_hbm.at[io_v.at[0]], s1)
        b0.wait()
        b1.wait()

    return k(x, pose3, poso3)


def _ffn_kernel(meta_ref, xs_ref, w1_ref, b1_ref, w2_ref, b2_ref, ys_ref):
    t = pl.program_id(0)

    @pl.when(t < meta_ref[NTILES])
    def _():
        xb = xs_ref[...].astype(jnp.bfloat16)
        h = jnp.dot(xb, w1_ref[0].astype(jnp.bfloat16),
                    preferred_element_type=jnp.float32) + b1_ref[0]
        h = 0.5 * h * (1.0 + lax.erf(h * 0.7071067811865476))
        ys_ref[...] = jnp.dot(h.astype(jnp.bfloat16),
                              w2_ref[0].astype(jnp.bfloat16),
                              preferred_element_type=jnp.float32) + b2_ref[0]


def _ffn(meta, xs, W1, b1, W2, b2):
    grid_spec = pltpu.PrefetchScalarGridSpec(
        num_scalar_prefetch=1,
        grid=(NTILES,),
        in_specs=[
            pl.BlockSpec((BT, D),
                         lambda t, m: (jnp.minimum(t, m[NTILES] - 1), 0)),
            pl.BlockSpec((1, D, H), lambda t, m: (m[t], 0, 0)),
            pl.BlockSpec((1, 1, H), lambda t, m: (m[t], 0, 0)),
            pl.BlockSpec((1, H, D), lambda t, m: (m[t], 0, 0)),
            pl.BlockSpec((1, 1, D), lambda t, m: (m[t], 0, 0)),
        ],
        out_specs=pl.BlockSpec((BT, D),
                               lambda t, m: (jnp.minimum(t, m[NTILES] - 1), 0)),
    )
    return pl.pallas_call(
        _ffn_kernel,
        grid_spec=grid_spec,
        out_shape=jax.ShapeDtypeStruct((S_PAD, D), jnp.float32),
    )(meta, xs, W1, b1.reshape(E, 1, H), W2, b2.reshape(E, 1, D))


def _sc_gather_y(ys, pose3, poso3):
    mesh = plsc.VectorSubcoreMesh(core_axis_name="c", subcore_axis_name="s")
    oty = jax.ShapeDtypeStruct((T, D), jnp.float32)

    @functools.partial(
        pl.kernel, mesh=mesh,
        out_type=[oty, oty],
        scratch_types=[
            pltpu.VMEM((1, TPW), jnp.int32),
            pltpu.VMEM((1, TPW), jnp.int32),
            pltpu.VMEM((TPW, D), jnp.float32),
            pltpu.VMEM((TPW, D), jnp.float32),
            pltpu.SemaphoreType.DMA,
            pltpu.SemaphoreType.DMA,
        ],
    )
    def k(ys_hbm, pose_hbm, poso_hbm, y0_hbm, y1_hbm, ie_v, io_v, r0_v, r1_v,
          s0, s1):
        wid = lax.axis_index("s") * NC + lax.axis_index("c")
        a0 = pltpu.async_copy(pose_hbm.at[wid], ie_v, s0)
        a1 = pltpu.async_copy(poso_hbm.at[wid], io_v, s1)
        a0.wait()
        a1.wait()
        b0 = pltpu.async_copy(ys_hbm.at[ie_v.at[0]], r0_v, s0)
        b1 = pltpu.async_copy(ys_hbm.at[io_v.at[0]], r1_v, s1)
        b0.wait()
        b1.wait()
        c0 = pltpu.async_copy(r0_v, y0_hbm.at[pl.ds(wid * TPW, TPW)], s0)
        c1 = pltpu.async_copy(r1_v, y1_hbm.at[pl.ds(wid * TPW, TPW)], s1)
        c0.wait()
        c1.wait()

    return k(ys, pose3, poso3)


def _combine_kernel(y0_ref, y1_ref, w0_ref, w1_ref, out_ref):
    out_ref[...] = y0_ref[...] * w0_ref[...] + y1_ref[...] * w1_ref[...]


def _combine(y0, y1, w0, w1):
    return pl.pallas_call(
        _combine_kernel,
        grid=(4,),
        in_specs=[
            pl.BlockSpec((T // 4, D), lambda i: (i, 0)),
            pl.BlockSpec((T // 4, D), lambda i: (i, 0)),
            pl.BlockSpec((T // 4, 1), lambda i: (i, 0)),
            pl.BlockSpec((T // 4, 1), lambda i: (i, 0)),
        ],
        out_specs=pl.BlockSpec((T // 4, D), lambda i: (i, 0)),
        out_shape=jax.ShapeDtypeStruct((T, D), jnp.float32),
    )(y0, y1, w0, w1)


def kernel(x, Wg, bg, W1, b1, W2, b2):
    pose, poso, w0, w1, meta = _gate(x, Wg, bg)
    pose3 = pose.reshape(NW, 1, TPW)
    poso3 = poso.reshape(NW, 1, TPW)
    xs = _sc_scatter_xs(x, pose3, poso3)
    ys = _ffn(meta.reshape(MLEN), xs, W1, b1, W2, b2)
    y0, y1 = _sc_gather_y(ys, pose3, poso3)
    return _combine(y0, y1, w0, w1)


# speedup vs baseline: 4.1862x; 1.0426x over previous
import functools

import jax
import jax.numpy as jnp
from jax import lax
from jax.experimental import pallas as pl
from jax.experimental.pallas import tpu as pltpu
from jax.experimental.pallas import tpu_sc as plsc

D = 768
H = 3072
E = 8
T = 2048
BT = 256
NTILES = 24
S_PAD = NTILES * BT
NC = 2
NS = 16
NW = NC * NS
TPW = T // NW
MLEN = 32


def _gate_kernel(x_ref, wg_ref, bg_ref,
                 pose_ref, poso_ref, w0_ref, w1_ref, meta_ref):
    logits = jnp.dot(x_ref[...].astype(jnp.bfloat16),
                     wg_ref[...].astype(jnp.bfloat16),
                     preferred_element_type=jnp.float32) + bg_ref[...]
    lane2 = lax.broadcasted_iota(jnp.int32, logits.shape, 1)
    v1 = jnp.max(logits, axis=-1, keepdims=True)
    i1 = jnp.min(jnp.where(logits == v1, lane2, E), axis=-1, keepdims=True)
    oh1 = lane2 == i1
    masked = jnp.where(oh1, -jnp.inf, logits)
    v2 = jnp.max(masked, axis=-1, keepdims=True)
    i2 = jnp.min(jnp.where(masked == v2, lane2, E), axis=-1, keepdims=True)
    oh2 = lane2 == i2
    lane16z = jnp.zeros((1, 16), jnp.float32)
    w0_ref[...] = jax.nn.sigmoid(v1 - v2) + lane16z
    w1_ref[...] = jax.nn.sigmoid(v2 - v1) + lane16z

    both = (oh1 | oh2).astype(jnp.bfloat16)
    rr = lax.broadcasted_iota(jnp.int32, (T, T), 0)
    cc = lax.broadcasted_iota(jnp.int32, (T, T), 1)
    ltri = (cc < rr).astype(jnp.bfloat16)
    excl = jnp.dot(ltri, both, preferred_element_type=jnp.float32)
    counts = jnp.sum(both.astype(jnp.float32), axis=0, keepdims=True)
    pc = jnp.ceil(counts / BT) * BT
    lane8f = lax.broadcasted_iota(jnp.int32, (1, E), 1).astype(jnp.float32)
    cum_pc = jnp.zeros((1, E), jnp.float32)
    for e in range(E):
        ce = jnp.sum(jnp.where(lane8f <= e, pc, 0.0))
        cum_pc = jnp.where(lane8f == e, ce, cum_pc)
    goff = cum_pc - pc

    def pick(mat, idx):
        return jnp.sum(jnp.where(lane2 == idx, mat, 0.0),
                       axis=-1, keepdims=True)

    pose_ref[...] = (pick(goff + excl, i1)).astype(jnp.int32)
    poso_ref[...] = (pick(goff + excl, i2)).astype(jnp.int32)

    lane32 = lax.broadcasted_iota(jnp.int32, (1, MLEN), 1)
    lane8 = lax.broadcasted_iota(jnp.int32, (1, E), 1).astype(jnp.float32)
    acc = jnp.zeros((1, MLEN), jnp.int32)
    for e in range(E):
        cum_e = jnp.sum(jnp.where(lane8 == e, cum_pc, 0.0))
        acc = acc + (cum_e <= (lane32 * BT).astype(jnp.float32)).astype(jnp.int32)
    te = jnp.minimum(acc, E - 1)
    used = (jnp.sum(jnp.where(lane8 == E - 1, cum_pc, 0.0)) / BT).astype(jnp.int32)
    meta_ref[...] = jnp.where(lane32 < NTILES, te, used)


def _gate(x, Wg, bg):
    return pl.pallas_call(
        _gate_kernel,
        grid=(1,),
        in_specs=[
            pl.BlockSpec((T, D), lambda i: (0, 0)),
            pl.BlockSpec((D, E), lambda i: (0, 0)),
            pl.BlockSpec((1, E), lambda i: (0, 0)),
        ],
        out_specs=[
            pl.BlockSpec((T, 1), lambda i: (0, 0)),
            pl.BlockSpec((T, 1), lambda i: (0, 0)),
            pl.BlockSpec((T, 16), lambda i: (0, 0)),
            pl.BlockSpec((T, 16), lambda i: (0, 0)),
            pl.BlockSpec((1, MLEN), lambda i: (0, 0)),
        ],
        out_shape=[
            jax.ShapeDtypeStruct((T, 1), jnp.int32),
            jax.ShapeDtypeStruct((T, 1), jnp.int32),
            jax.ShapeDtypeStruct((T, 16), jnp.float32),
            jax.ShapeDtypeStruct((T, 16), jnp.float32),
            jax.ShapeDtypeStruct((1, MLEN), jnp.int32),
        ],
    )(x, Wg, bg.reshape(1, E))


def _sc_scatter_xs(x, pose3, poso3):
    mesh = plsc.VectorSubcoreMesh(core_axis_name="c", subcore_axis_name="s")

    @functools.partial(
        pl.kernel, mesh=mesh,
        out_type=jax.ShapeDtypeStruct((S_PAD, D), jnp.float32),
        scratch_types=[
            pltpu.VMEM((1, TPW), jnp.int32),
            pltpu.VMEM((1, TPW), jnp.int32),
            pltpu.VMEM((TPW, D), jnp.float32),
            pltpu.SemaphoreType.DMA,
            pltpu.SemaphoreType.DMA,
            pltpu.SemaphoreType.DMA,
        ],
    )
    def k(x_hbm, pose_hbm, poso_hbm, xs_hbm, ie_v, io_v, rows_v, s0, s1, s2):
        wid = lax.axis_index("s") * NC + lax.axis_index("c")
        a0 = pltpu.async_copy(pose_hbm.at[wid], ie_v, s0)
        a1 = pltpu.async_copy(poso_hbm.at[wid], io_v, s1)
        a2 = pltpu.async_copy(x_hbm.at[pl.ds(wid * TPW, TPW)], rows_v, s2)
        a0.wait()
        a1.wait()
        a2.wait()
        b0 = pltpu.async_copy(rows_v, xs_hbm.at[ie_v.at[0]], s0)
        b1 = pltpu.async_copy(rows_v, xs_hbm.at[io_v.at[0]], s1)
        b0.wait()
        b1.wait()

    return k(x, pose3, poso3)


def _ffn_kernel(meta_ref, xs_ref, w1_ref, b1_ref, w2_ref, b2_ref, ys_ref):
    t = pl.program_id(0)

    @pl.when(t < meta_ref[NTILES])
    def _():
        xb = xs_ref[...].astype(jnp.bfloat16)
        h = jnp.dot(xb, w1_ref[0].astype(jnp.bfloat16),
                    preferred_element_type=jnp.float32) + b1_ref[0]
        h = 0.5 * h * (1.0 + lax.erf(h * 0.7071067811865476))
        ys_ref[...] = jnp.dot(h.astype(jnp.bfloat16),
                              w2_ref[0].astype(jnp.bfloat16),
                              preferred_element_type=jnp.float32) + b2_ref[0]


def _ffn(meta, xs, W1, b1, W2, b2):
    grid_spec = pltpu.PrefetchScalarGridSpec(
        num_scalar_prefetch=1,
        grid=(NTILES,),
        in_specs=[
            pl.BlockSpec((BT, D),
                         lambda t, m: (jnp.minimum(t, m[NTILES] - 1), 0)),
            pl.BlockSpec((1, D, H), lambda t, m: (m[t], 0, 0)),
            pl.BlockSpec((1, 1, H), lambda t, m: (m[t], 0, 0)),
            pl.BlockSpec((1, H, D), lambda t, m: (m[t], 0, 0)),
            pl.BlockSpec((1, 1, D), lambda t, m: (m[t], 0, 0)),
        ],
        out_specs=pl.BlockSpec((BT, D),
                               lambda t, m: (jnp.minimum(t, m[NTILES] - 1), 0)),
    )
    return pl.pallas_call(
        _ffn_kernel,
        grid_spec=grid_spec,
        out_shape=jax.ShapeDtypeStruct((S_PAD, D), jnp.float32),
    )(meta, xs, W1, b1.reshape(E, 1, H), W2, b2.reshape(E, 1, D))


def _sc_gather_combine(ys, pose3, poso3, w0r, w1r):
    mesh = plsc.VectorSubcoreMesh(core_axis_name="c", subcore_axis_name="s")

    @functools.partial(
        pl.kernel, mesh=mesh,
        out_type=jax.ShapeDtypeStruct((T, D), jnp.float32),
        scratch_types=[
            pltpu.VMEM((1, TPW), jnp.int32),
            pltpu.VMEM((1, TPW), jnp.int32),
            pltpu.VMEM((TPW, D), jnp.float32),
            pltpu.VMEM((TPW, D), jnp.float32),
            pltpu.VMEM((TPW, 16), jnp.float32),
            pltpu.VMEM((TPW, 16), jnp.float32),
            pltpu.SemaphoreType.DMA,
            pltpu.SemaphoreType.DMA,
            pltpu.SemaphoreType.DMA,
            pltpu.SemaphoreType.DMA,
        ],
    )
    def k(ys_hbm, pose_hbm, poso_hbm, w0_hbm, w1_hbm, out_hbm,
          ie_v, io_v, r0_v, r1_v, wv0, wv1, s0, s1, s2, s3):
        wid = lax.axis_index("s") * NC + lax.axis_index("c")
        a0 = pltpu.async_copy(pose_hbm.at[wid], ie_v, s0)
        a1 = pltpu.async_copy(poso_hbm.at[wid], io_v, s1)
        a2 = pltpu.async_copy(w0_hbm.at[pl.ds(wid * TPW, TPW)], wv0, s2)
        a3 = pltpu.async_copy(w1_hbm.at[pl.ds(wid * TPW, TPW)], wv1, s3)
        a0.wait()
        a1.wait()
        b0 = pltpu.async_copy(ys_hbm.at[ie_v.at[0]], r0_v, s0)
        b1 = pltpu.async_copy(ys_hbm.at[io_v.at[0]], r1_v, s1)
        a2.wait()
        a3.wait()
        b0.wait()
        b1.wait()

        @pl.loop(0, TPW)
        def _(i):
            w0c = wv0.at[pl.ds(i, 1), :][...]
            w1c = wv1.at[pl.ds(i, 1), :][...]
            for c in range(0, D, 16):
                slc = (pl.ds(i, 1), pl.ds(c, 16))
                r0_v.at[slc[0], slc[1]][...] = (
                    r0_v.at[slc[0], slc[1]][...] * w0c
                    + r1_v.at[slc[0], slc[1]][...] * w1c)

        pltpu.sync_copy(r0_v, out_hbm.at[pl.ds(wid * TPW, TPW)])

    return k(ys, pose3, poso3, w0r, w1r)


def kernel(x, Wg, bg, W1, b1, W2, b2):
    pose, poso, w0, w1, meta = _gate(x, Wg, bg)
    pose3 = pose.reshape(NW, 1, TPW)
    poso3 = poso.reshape(NW, 1, TPW)
    xs = _sc_scatter_xs(x, pose3, poso3)
    ys = _ffn(meta.reshape(MLEN), xs, W1, b1, W2, b2)
    return _sc_gather_combine(ys, pose3, poso3, w0, w1)


# speedup vs baseline: 4.2194x; 1.0079x over previous
import functools

import jax
import jax.numpy as jnp
from jax import lax
from jax.experimental import pallas as pl
from jax.experimental.pallas import tpu as pltpu
from jax.experimental.pallas import tpu_sc as plsc

D = 768
H = 3072
E = 8
T = 2048
BT = 256
NTILES = 24
S_PAD = NTILES * BT
NC = 2
NS = 16
NW = NC * NS
TPW = T // NW
MLEN = 32


def _gate_kernel(x_ref, wg_ref, bg_ref,
                 pose_ref, poso_ref, w0_ref, w1_ref, meta_ref):
    logits = jnp.dot(x_ref[...].astype(jnp.bfloat16),
                     wg_ref[...].astype(jnp.bfloat16),
                     preferred_element_type=jnp.float32) + bg_ref[...]
    lane2 = lax.broadcasted_iota(jnp.int32, logits.shape, 1)
    v1 = jnp.max(logits, axis=-1, keepdims=True)
    i1 = jnp.min(jnp.where(logits == v1, lane2, E), axis=-1, keepdims=True)
    oh1 = lane2 == i1
    masked = jnp.where(oh1, -jnp.inf, logits)
    v2 = jnp.max(masked, axis=-1, keepdims=True)
    i2 = jnp.min(jnp.where(masked == v2, lane2, E), axis=-1, keepdims=True)
    oh2 = lane2 == i2
    lane16z = jnp.zeros((1, 16), jnp.float32)
    w0_ref[...] = jax.nn.sigmoid(v1 - v2) + lane16z
    w1_ref[...] = jax.nn.sigmoid(v2 - v1) + lane16z

    both = (oh1 | oh2).astype(jnp.bfloat16)
    rr = lax.broadcasted_iota(jnp.int32, (T, T), 0)
    cc = lax.broadcasted_iota(jnp.int32, (T, T), 1)
    ltri = (cc < rr).astype(jnp.bfloat16)
    excl = jnp.dot(ltri, both, preferred_element_type=jnp.float32)
    counts = jnp.sum(both.astype(jnp.float32), axis=0, keepdims=True)
    pc = jnp.ceil(counts / BT) * BT
    lane8f = lax.broadcasted_iota(jnp.int32, (1, E), 1).astype(jnp.float32)
    cum_pc = jnp.zeros((1, E), jnp.float32)
    for e in range(E):
        ce = jnp.sum(jnp.where(lane8f <= e, pc, 0.0))
        cum_pc = jnp.where(lane8f == e, ce, cum_pc)
    goff = cum_pc - pc

    def pick(mat, idx):
        return jnp.sum(jnp.where(lane2 == idx, mat, 0.0),
                       axis=-1, keepdims=True)

    pose_ref[...] = (pick(goff + excl, i1)).astype(jnp.int32)
    poso_ref[...] = (pick(goff + excl, i2)).astype(jnp.int32)

    lane32 = lax.broadcasted_iota(jnp.int32, (1, MLEN), 1)
    lane8 = lax.broadcasted_iota(jnp.int32, (1, E), 1).astype(jnp.float32)
    acc = jnp.zeros((1, MLEN), jnp.int32)
    for e in range(E):
        cum_e = jnp.sum(jnp.where(lane8 == e, cum_pc, 0.0))
        acc = acc + (cum_e <= (lane32 * BT).astype(jnp.float32)).astype(jnp.int32)
    te = jnp.minimum(acc, E - 1)
    used = (jnp.sum(jnp.where(lane8 == E - 1, cum_pc, 0.0)) / BT).astype(jnp.int32)
    meta_ref[...] = jnp.where(lane32 < NTILES, te, used)


def _gate(x, Wg, bg):
    return pl.pallas_call(
        _gate_kernel,
        grid=(1,),
        in_specs=[
            pl.BlockSpec((T, D), lambda i: (0, 0)),
            pl.BlockSpec((D, E), lambda i: (0, 0)),
            pl.BlockSpec((1, E), lambda i: (0, 0)),
        ],
        out_specs=[
            pl.BlockSpec((T, 1), lambda i: (0, 0)),
            pl.BlockSpec((T, 1), lambda i: (0, 0)),
            pl.BlockSpec((T, 16), lambda i: (0, 0)),
            pl.BlockSpec((T, 16), lambda i: (0, 0)),
            pl.BlockSpec((1, MLEN), lambda i: (0, 0)),
        ],
        out_shape=[
            jax.ShapeDtypeStruct((T, 1), jnp.int32),
            jax.ShapeDtypeStruct((T, 1), jnp.int32),
            jax.ShapeDtypeStruct((T, 16), jnp.float32),
            jax.ShapeDtypeStruct((T, 16), jnp.float32),
            jax.ShapeDtypeStruct((1, MLEN), jnp.int32),
        ],
    )(x, Wg, bg.reshape(1, E))


def _sc_scatter_xs(x, pose3, poso3):
    mesh = plsc.VectorSubcoreMesh(core_axis_name="c", subcore_axis_name="s")

    @functools.partial(
        pl.kernel, mesh=mesh,
        out_type=jax.ShapeDtypeStruct((S_PAD, D), jnp.float32),
        scratch_types=[
            pltpu.VMEM((1, TPW), jnp.int32),
            pltpu.VMEM((1, TPW), jnp.int32),
            pltpu.VMEM((TPW, D), jnp.float32),
            pltpu.SemaphoreType.DMA,
            pltpu.SemaphoreType.DMA,
            pltpu.SemaphoreType.DMA,
        ],
    )
    def k(x_hbm, pose_hbm, poso_hbm, xs_hbm, ie_v, io_v, rows_v, s0, s1, s2):
        wid = lax.axis_index("s") * NC + lax.axis_index("c")
        a0 = pltpu.async_copy(pose_hbm.at[wid], ie_v, s0)
        a1 = pltpu.async_copy(poso_hbm.at[wid], io_v, s1)
        a2 = pltpu.async_copy(x_hbm.at[pl.ds(wid * TPW, TPW)], rows_v, s2)
        a0.wait()
        a1.wait()
        a2.wait()
        b0 = pltpu.async_copy(rows_v, xs_hbm.at[ie_v.at[0]], s0)
        b1 = pltpu.async_copy(rows_v, xs_hbm.at[io_v.at[0]], s1)
        b0.wait()
        b1.wait()

    return k(x, pose3, poso3)


def _ffn_kernel(meta_ref, xs_ref, w1_ref, b1_ref, w2_ref, b2_ref, ys_ref):
    t = pl.program_id(0)

    @pl.when(t < meta_ref[NTILES])
    def _():
        xb = xs_ref[...].astype(jnp.bfloat16)
        h = jnp.dot(xb, w1_ref[0].astype(jnp.bfloat16),
                    preferred_element_type=jnp.float32) + b1_ref[0]
        h = 0.5 * h * (1.0 + lax.erf(h * 0.7071067811865476))
        ys_ref[...] = jnp.dot(h.astype(jnp.bfloat16),
                              w2_ref[0].astype(jnp.bfloat16),
                              preferred_element_type=jnp.float32) + b2_ref[0]


def _ffn(meta, xs, W1, b1, W2, b2):
    grid_spec = pltpu.PrefetchScalarGridSpec(
        num_scalar_prefetch=1,
        grid=(NTILES,),
        in_specs=[
            pl.BlockSpec((BT, D),
                         lambda t, m: (jnp.minimum(t, m[NTILES] - 1), 0)),
            pl.BlockSpec((1, D, H), lambda t, m: (m[t], 0, 0)),
            pl.BlockSpec((1, 1, H), lambda t, m: (m[t], 0, 0)),
            pl.BlockSpec((1, H, D), lambda t, m: (m[t], 0, 0)),
            pl.BlockSpec((1, 1, D), lambda t, m: (m[t], 0, 0)),
        ],
        out_specs=pl.BlockSpec((BT, D),
                               lambda t, m: (jnp.minimum(t, m[NTILES] - 1), 0)),
    )
    return pl.pallas_call(
        _ffn_kernel,
        grid_spec=grid_spec,
        out_shape=jax.ShapeDtypeStruct((S_PAD, D), jnp.float32),
    )(meta, xs, W1, b1.reshape(E, 1, H), W2, b2.reshape(E, 1, D))


def _sc_gather_combine(ys, pose3, poso3, w0r, w1r):
    mesh = plsc.VectorSubcoreMesh(core_axis_name="c", subcore_axis_name="s")

    @functools.partial(
        pl.kernel, mesh=mesh,
        out_type=jax.ShapeDtypeStruct((T, D), jnp.float32),
        scratch_types=[
            pltpu.VMEM((1, TPW), jnp.int32),
            pltpu.VMEM((1, TPW), jnp.int32),
            pltpu.VMEM((TPW, D), jnp.float32),
            pltpu.VMEM((TPW, D), jnp.float32),
            pltpu.VMEM((TPW, 16), jnp.float32),
            pltpu.VMEM((TPW, 16), jnp.float32),
            pltpu.SemaphoreType.DMA,
            pltpu.SemaphoreType.DMA,
            pltpu.SemaphoreType.DMA,
            pltpu.SemaphoreType.DMA,
        ],
    )
    def k(ys_hbm, pose_hbm, poso_hbm, w0_hbm, w1_hbm, out_hbm,
          ie_v, io_v, r0_v, r1_v, wv0, wv1, s0, s1, s2, s3):
        wid = lax.axis_index("s") * NC + lax.axis_index("c")
        hw = TPW // 2
        a0 = pltpu.async_copy(pose_hbm.at[wid], ie_v, s0)
        a1 = pltpu.async_copy(poso_hbm.at[wid], io_v, s1)
        a2 = pltpu.async_copy(w0_hbm.at[pl.ds(wid * TPW, TPW)], wv0, s2)
        a3 = pltpu.async_copy(w1_hbm.at[pl.ds(wid * TPW, TPW)], wv1, s3)
        a0.wait()
        a1.wait()
        b0 = pltpu.async_copy(ys_hbm.at[ie_v.at[0, pl.ds(0, hw)]],
                              r0_v.at[pl.ds(0, hw)], s0)
        b1 = pltpu.async_copy(ys_hbm.at[io_v.at[0, pl.ds(0, hw)]],
                              r1_v.at[pl.ds(0, hw)], s1)
        a2.wait()
        a3.wait()
        b0.wait()
        b1.wait()
        c0 = pltpu.async_copy(ys_hbm.at[ie_v.at[0, pl.ds(hw, hw)]],
                              r0_v.at[pl.ds(hw, hw)], s0)
        c1 = pltpu.async_copy(ys_hbm.at[io_v.at[0, pl.ds(hw, hw)]],
                              r1_v.at[pl.ds(hw, hw)], s1)

        def combine(base):
            @pl.loop(base, base + hw)
            def _(i):
                w0c = wv0.at[pl.ds(i, 1), :][...]
                w1c = wv1.at[pl.ds(i, 1), :][...]
                for c in range(0, D, 16):
                    slc = (pl.ds(i, 1), pl.ds(c, 16))
                    r0_v.at[slc[0], slc[1]][...] = (
                        r0_v.at[slc[0], slc[1]][...] * w0c
                        + r1_v.at[slc[0], slc[1]][...] * w1c)

        combine(0)
        d0 = pltpu.async_copy(r0_v.at[pl.ds(0, hw)],
                              out_hbm.at[pl.ds(wid * TPW, hw)], s2)
        c0.wait()
        c1.wait()
        combine(hw)
        d1 = pltpu.async_copy(r0_v.at[pl.ds(hw, hw)],
                              out_hbm.at[pl.ds(wid * TPW + hw, hw)], s3)
        d0.wait()
        d1.wait()

    return k(ys, pose3, poso3, w0r, w1r)


def kernel(x, Wg, bg, W1, b1, W2, b2):
    pose, poso, w0, w1, meta = _gate(x, Wg, bg)
    pose3 = pose.reshape(NW, 1, TPW)
    poso3 = poso.reshape(NW, 1, TPW)
    xs = _sc_scatter_xs(x, pose3, poso3)
    ys = _ffn(meta.reshape(MLEN), xs, W1, b1, W2, b2)
    return _sc_gather_combine(ys, pose3, poso3, w0, w1)
